# baseline (speedup 1.0000x reference)
import functools
import math

import jax
import jax.numpy as jnp
from jax.experimental import pallas as pl

GS = 128
HEADS = 16
DH = 64
CROSS = math.log(0.125)
BETA_LF = 0.675
BETA_MF = 0.225
BETA_HF = 0.1


def _qkv_pool_kernel(s, x_ref, srow_ref, pm_ref, st_ref, lt_ref, w_ref,
                     qkv_ref, pl_ref):
    xb = x_ref[0]
    if s == 1:
        px = xb
    else:
        R = GS * s
        w = jnp.clip(srow_ref[0], 1e-6, None)
        wm = pm_ref[:] * jnp.broadcast_to(w, (GS, R))
        den = jnp.sum(wm, axis=1, keepdims=True)
        wn = (wm / den).astype(jnp.bfloat16)
        px = jnp.dot(wn, xb, preferred_element_type=jnp.float32
                     ).astype(jnp.bfloat16)
        sg = st_ref[0]
        lg = lt_ref[0]
        if s == 2:
            plab = jnp.where(sg[0:1] >= sg[1:2], lg[0:1], lg[1:2])
        else:
            m01 = jnp.maximum(sg[0:1], sg[1:2])
            l01 = jnp.where(sg[0:1] >= sg[1:2], lg[0:1], lg[1:2])
            m23 = jnp.maximum(sg[2:3], sg[3:4])
            l23 = jnp.where(sg[2:3] >= sg[3:4], lg[2:3], lg[3:4])
            plab = jnp.where(m01 >= m23, l01, l23)
        pl_ref[0] = plab
    qkv_ref[0] = jnp.dot(px, w_ref[:],
                         preferred_element_type=jnp.float32
                         ).astype(jnp.bfloat16)


def _attn_kernel(ng, fuse, *refs):
    if fuse:
        (q_ref, ks_ref, kn_ref, vs_ref, vn_ref, ls_ref, ln_ref,
         amf_ref, alf_ref, xres_ref, wp_ref, out_ref) = refs
    else:
        (q_ref, ks_ref, kn_ref, vs_ref, vn_ref, ls_ref, ln_ref,
         out_ref) = refs
    scale = 1.0 / math.sqrt(DH)
    q = (q_ref[0].astype(jnp.float32) * scale).astype(jnp.bfloat16)
    ks = ks_ref[0]
    kn = kn_ref[0]
    vs = vs_ref[0]
    vn = vn_ref[0]
    kl = jnp.concatenate([ls_ref[0], ln_ref[0]], axis=1)
    qlT = ls_ref[0].T
    bias = jnp.where(qlT == kl, 0.0, CROSS)
    outs = []
    for h in range(HEADS):
        sl = slice(h * DH, (h + 1) * DH)
        qh = q[:, sl]
        lg1 = jax.lax.dot_general(qh, ks[:, sl], (((1,), (1,)), ((), ())),
                                  preferred_element_type=jnp.float32)
        lg2 = jax.lax.dot_general(qh, kn[:, sl], (((1,), (1,)), ((), ())),
                                  preferred_element_type=jnp.float32)
        p = jnp.exp(jnp.concatenate([lg1, lg2], axis=1) + bias)
        attn = (p / jnp.sum(p, axis=-1, keepdims=True)).astype(jnp.bfloat16)
        o = jnp.dot(attn[:, :GS], vs[:, sl], preferred_element_type=jnp.float32)
        o += jnp.dot(attn[:, GS:], vn[:, sl], preferred_element_type=jnp.float32)
        outs.append(o)
    a = jnp.concatenate(outs, axis=1)
    if fuse:
        amf = amf_ref[0].astype(jnp.float32)
        up2 = jnp.broadcast_to(amf[:, None, :], (64, 2, amf.shape[-1])
                               ).reshape(128, amf.shape[-1])
        alf = alf_ref[0].astype(jnp.float32)
        up4 = jnp.broadcast_to(alf[:, None, :], (32, 4, alf.shape[-1])
                               ).reshape(128, alf.shape[-1])
        fused = BETA_HF * a + BETA_MF * up2 + BETA_LF * up4
        out_ref[0] = jnp.dot(fused.astype(jnp.bfloat16), wp_ref[:],
                             preferred_element_type=jnp.float32) + xres_ref[0]
    else:
        out_ref[0] = a.astype(jnp.bfloat16)


def _run_qkv(s, x_bf, scores, labels, wqkv, interpret=False):
    B, N, C = x_bf.shape
    np_ = N // s
    ng = np_ // GS
    R = GS * s
    srow = scores.reshape(B * ng, 1, R)
    st = scores.reshape(B * ng, GS, s).transpose(0, 2, 1)
    lt = labels.reshape(B * ng, GS, s).transpose(0, 2, 1).astype(jnp.int32)
    pmask = (jnp.arange(R)[None, :] // s == jnp.arange(GS)[:, None]
             ).astype(jnp.float32)
    grid = (B, ng)
    kern = functools.partial(_qkv_pool_kernel, s)
    qkv, plab = pl.pallas_call(
        kern,
        grid=grid,
        in_specs=[
            pl.BlockSpec((1, R, C), lambda b, i: (b, i, 0)),
            pl.BlockSpec((1, 1, R), lambda b, i, ng=ng: (b * ng + i, 0, 0)),
            pl.BlockSpec((GS, R), lambda b, i: (0, 0)),
            pl.BlockSpec((1, s, GS), lambda b, i, ng=ng: (b * ng + i, 0, 0)),
            pl.BlockSpec((1, s, GS), lambda b, i, ng=ng: (b * ng + i, 0, 0)),
            pl.BlockSpec((C, 3 * C), lambda b, i: (0, 0)),
        ],
        out_specs=[
            pl.BlockSpec((1, GS, 3 * C), lambda b, i: (b, i, 0)),
            pl.BlockSpec((1, 1, GS), lambda b, i, ng=ng: (b * ng + i, 0, 0)),
        ],
        out_shape=[
            jax.ShapeDtypeStruct((B, np_, 3 * C), jnp.bfloat16),
            jax.ShapeDtypeStruct((B * ng, 1, GS), jnp.int32),
        ],
        interpret=interpret,
    )(x_bf, srow, pmask, st, lt, wqkv)
    return qkv, plab


def _run_attn(s, qkv, plab, fuse_args, interpret=False):
    B, np_, C3 = qkv.shape
    C = C3 // 3
    ng = np_ // GS
    labs = plab.reshape(B * ng, 1, GS)
    nxt = lambda i: jnp.minimum(i + 1, ng - 1)
    in_specs = [
        pl.BlockSpec((1, GS, C), lambda b, i: (b, i, 0)),
        pl.BlockSpec((1, GS, C), lambda b, i: (b, i, 1)),
        pl.BlockSpec((1, GS, C), lambda b, i: (b, nxt(i), 1)),
        pl.BlockSpec((1, GS, C), lambda b, i: (b, i, 2)),
        pl.BlockSpec((1, GS, C), lambda b, i: (b, nxt(i), 2)),
        pl.BlockSpec((1, 1, GS), lambda b, i, ng=ng: (b * ng + i, 0, 0)),
        pl.BlockSpec((1, 1, GS), lambda b, i, ng=ng: (b * ng + nxt(i), 0, 0)),
    ]
    args = [qkv, qkv, qkv, qkv, qkv, labs, labs]
    if fuse_args is not None:
        amf, alf, x, wp = fuse_args
        in_specs += [
            pl.BlockSpec((1, GS // 2, C), lambda b, i: (b, i, 0)),
            pl.BlockSpec((1, GS // 4, C), lambda b, i: (b, i, 0)),
            pl.BlockSpec((1, GS, C), lambda b, i: (b, i, 0)),
            pl.BlockSpec((C, C), lambda b, i: (0, 0)),
        ]
        args += [amf, alf, x, wp]
    kern = functools.partial(_attn_kernel, ng, fuse_args is not None)
    out = pl.pallas_call(
        kern,
        grid=(B, ng),
        in_specs=in_specs,
        out_specs=pl.BlockSpec((1, GS, C), lambda b, i: (b, i, 0)),
        out_shape=jax.ShapeDtypeStruct(
            (B, np_, C), jnp.float32 if fuse_args is not None else jnp.bfloat16),
        interpret=interpret,
    )(*args)
    return out


def _impl(x, labels, scores, Wq_hf, Wk_hf, Wv_hf, Wq_mf, Wk_mf, Wv_mf,
          Wq_lf, Wk_lf, Wv_lf, Wp, interpret=False):
    B, N, C = x.shape
    labels = labels.astype(jnp.int32)
    x_bf = x.astype(jnp.bfloat16)
    w_lf = jnp.concatenate([Wq_lf, Wk_lf, Wv_lf], axis=1).astype(jnp.bfloat16)
    w_mf = jnp.concatenate([Wq_mf, Wk_mf, Wv_mf], axis=1).astype(jnp.bfloat16)
    w_hf = jnp.concatenate([Wq_hf, Wk_hf, Wv_hf], axis=1).astype(jnp.bfloat16)
    Wp = Wp.astype(jnp.bfloat16)

    qkv_lf, pl_lf = _run_qkv(4, x_bf, scores, labels, w_lf, interpret)
    a_lf = _run_attn(4, qkv_lf, pl_lf, None, interpret)

    qkv_mf, pl_mf = _run_qkv(2, x_bf, scores, labels, w_mf, interpret)
    a_mf = _run_attn(2, qkv_mf, pl_mf, None, interpret)

    qkv_hf, _ = _run_qkv(1, x_bf, scores, labels, w_hf, interpret)
    ng_hf = N // GS
    pl_hf = labels.reshape(B * ng_hf, 1, GS)
    out = _run_attn(1, qkv_hf, pl_hf, (a_mf, a_lf, x, Wp), interpret)
    return out


def kernel(x, labels, scores, Wq_hf, Wk_hf, Wv_hf, Wq_mf, Wk_mf, Wv_mf,
           Wq_lf, Wk_lf, Wv_lf, Wp):
    return _impl(x, labels, scores, Wq_hf, Wk_hf, Wv_hf, Wq_mf, Wk_mf,
                 Wv_mf, Wq_lf, Wk_lf, Wv_lf, Wp)


# speedup vs baseline: 1.8488x; 1.8488x over previous
import functools
import math

import jax
import jax.numpy as jnp
from jax.experimental import pallas as pl

GS = 128
HEADS = 16
DH = 64
CROSS = math.log(0.125)
BETA_LF = 0.675
BETA_MF = 0.225
BETA_HF = 0.1


def _qkv_pool_kernel(s, x_ref, srow_ref, pm_ref, st_ref, lt_ref, w_ref,
                     qkv_ref, pl_ref):
    xb = x_ref[0]
    if s == 1:
        px = xb
    else:
        R = GS * s
        w = jnp.clip(srow_ref[0], 1e-6, None)
        wm = pm_ref[:] * jnp.broadcast_to(w, (GS, R))
        den = jnp.sum(wm, axis=1, keepdims=True)
        wn = (wm / den).astype(jnp.bfloat16)
        px = jnp.dot(wn, xb, preferred_element_type=jnp.float32
                     ).astype(jnp.bfloat16)
        sg = st_ref[0]
        lg = lt_ref[0]
        if s == 2:
            plab = jnp.where(sg[0:1] >= sg[1:2], lg[0:1], lg[1:2])
        else:
            m01 = jnp.maximum(sg[0:1], sg[1:2])
            l01 = jnp.where(sg[0:1] >= sg[1:2], lg[0:1], lg[1:2])
            m23 = jnp.maximum(sg[2:3], sg[3:4])
            l23 = jnp.where(sg[2:3] >= sg[3:4], lg[2:3], lg[3:4])
            plab = jnp.where(m01 >= m23, l01, l23)
        pl_ref[0] = plab
    qkv_ref[0] = jnp.dot(px, w_ref[:],
                         preferred_element_type=jnp.float32
                         ).astype(jnp.bfloat16)


def _attn_kernel(ng, fuse, *refs):
    if fuse:
        (q_ref, ks_ref, kn_ref, vs_ref, vn_ref, ls_ref, ln_ref,
         amf_ref, alf_ref, xres_ref, wp_ref, out_ref) = refs
    else:
        (q_ref, ks_ref, kn_ref, vs_ref, vn_ref, ls_ref, ln_ref,
         out_ref) = refs
    scale = 1.0 / math.sqrt(DH)
    q = (q_ref[0].astype(jnp.float32) * scale).astype(jnp.bfloat16)
    k = jnp.concatenate([ks_ref[0], kn_ref[0]], axis=0)
    v = jnp.concatenate([vs_ref[0], vn_ref[0]], axis=0)
    kl = jnp.concatenate([ls_ref[0], ln_ref[0]], axis=1)
    qlT = ls_ref[0].T
    bias = jnp.where(qlT == kl, 0.0, CROSS)
    outs = []
    for h in range(HEADS):
        sl = slice(h * DH, (h + 1) * DH)
        lg = jax.lax.dot_general(q[:, sl], k[:, sl], (((1,), (1,)), ((), ())),
                                 preferred_element_type=jnp.float32)
        p = jnp.exp(lg + bias)
        s = jnp.sum(p, axis=-1, keepdims=True)
        o = jnp.dot(p.astype(jnp.bfloat16), v[:, sl],
                    preferred_element_type=jnp.float32)
        outs.append(o / s)
    a = jnp.concatenate(outs, axis=1)
    if fuse:
        amf = amf_ref[0].astype(jnp.float32)
        up2 = jnp.broadcast_to(amf[:, None, :], (64, 2, amf.shape[-1])
                               ).reshape(128, amf.shape[-1])
        alf = alf_ref[0].astype(jnp.float32)
        up4 = jnp.broadcast_to(alf[:, None, :], (32, 4, alf.shape[-1])
                               ).reshape(128, alf.shape[-1])
        fused = BETA_HF * a + BETA_MF * up2 + BETA_LF * up4
        out_ref[0] = jnp.dot(fused.astype(jnp.bfloat16), wp_ref[:],
                             preferred_element_type=jnp.float32) + xres_ref[0]
    else:
        out_ref[0] = a.astype(jnp.bfloat16)


def _run_qkv(s, x_bf, scores, labels, wqkv, interpret=False):
    B, N, C = x_bf.shape
    np_ = N // s
    ng = np_ // GS
    R = GS * s
    srow = scores.reshape(B * ng, 1, R)
    st = scores.reshape(B * ng, GS, s).transpose(0, 2, 1)
    lt = labels.reshape(B * ng, GS, s).transpose(0, 2, 1).astype(jnp.int32)
    pmask = (jnp.arange(R)[None, :] // s == jnp.arange(GS)[:, None]
             ).astype(jnp.float32)
    grid = (B, ng)
    kern = functools.partial(_qkv_pool_kernel, s)
    qkv, plab = pl.pallas_call(
        kern,
        grid=grid,
        in_specs=[
            pl.BlockSpec((1, R, C), lambda b, i: (b, i, 0)),
            pl.BlockSpec((1, 1, R), lambda b, i, ng=ng: (b * ng + i, 0, 0)),
            pl.BlockSpec((GS, R), lambda b, i: (0, 0)),
            pl.BlockSpec((1, s, GS), lambda b, i, ng=ng: (b * ng + i, 0, 0)),
            pl.BlockSpec((1, s, GS), lambda b, i, ng=ng: (b * ng + i, 0, 0)),
            pl.BlockSpec((C, 3 * C), lambda b, i: (0, 0)),
        ],
        out_specs=[
            pl.BlockSpec((1, GS, 3 * C), lambda b, i: (b, i, 0)),
            pl.BlockSpec((1, 1, GS), lambda b, i, ng=ng: (b * ng + i, 0, 0)),
        ],
        out_shape=[
            jax.ShapeDtypeStruct((B, np_, 3 * C), jnp.bfloat16),
            jax.ShapeDtypeStruct((B * ng, 1, GS), jnp.int32),
        ],
        interpret=interpret,
    )(x_bf, srow, pmask, st, lt, wqkv)
    return qkv, plab


def _run_attn(s, qkv, plab, fuse_args, interpret=False):
    B, np_, C3 = qkv.shape
    C = C3 // 3
    ng = np_ // GS
    labs = plab.reshape(B * ng, 1, GS)
    nxt = lambda i: jnp.minimum(i + 1, ng - 1)
    in_specs = [
        pl.BlockSpec((1, GS, C), lambda b, i: (b, i, 0)),
        pl.BlockSpec((1, GS, C), lambda b, i: (b, i, 1)),
        pl.BlockSpec((1, GS, C), lambda b, i: (b, nxt(i), 1)),
        pl.BlockSpec((1, GS, C), lambda b, i: (b, i, 2)),
        pl.BlockSpec((1, GS, C), lambda b, i: (b, nxt(i), 2)),
        pl.BlockSpec((1, 1, GS), lambda b, i, ng=ng: (b * ng + i, 0, 0)),
        pl.BlockSpec((1, 1, GS), lambda b, i, ng=ng: (b * ng + nxt(i), 0, 0)),
    ]
    args = [qkv, qkv, qkv, qkv, qkv, labs, labs]
    if fuse_args is not None:
        amf, alf, x, wp = fuse_args
        in_specs += [
            pl.BlockSpec((1, GS // 2, C), lambda b, i: (b, i, 0)),
            pl.BlockSpec((1, GS // 4, C), lambda b, i: (b, i, 0)),
            pl.BlockSpec((1, GS, C), lambda b, i: (b, i, 0)),
            pl.BlockSpec((C, C), lambda b, i: (0, 0)),
        ]
        args += [amf, alf, x, wp]
    kern = functools.partial(_attn_kernel, ng, fuse_args is not None)
    out = pl.pallas_call(
        kern,
        grid=(B, ng),
        in_specs=in_specs,
        out_specs=pl.BlockSpec((1, GS, C), lambda b, i: (b, i, 0)),
        out_shape=jax.ShapeDtypeStruct(
            (B, np_, C), jnp.float32 if fuse_args is not None else jnp.bfloat16),
        interpret=interpret,
    )(*args)
    return out


def _impl(x, labels, scores, Wq_hf, Wk_hf, Wv_hf, Wq_mf, Wk_mf, Wv_mf,
          Wq_lf, Wk_lf, Wv_lf, Wp, interpret=False):
    B, N, C = x.shape
    labels = labels.astype(jnp.int32)
    x_bf = x.astype(jnp.bfloat16)
    w_lf = jnp.concatenate([Wq_lf, Wk_lf, Wv_lf], axis=1).astype(jnp.bfloat16)
    w_mf = jnp.concatenate([Wq_mf, Wk_mf, Wv_mf], axis=1).astype(jnp.bfloat16)
    w_hf = jnp.concatenate([Wq_hf, Wk_hf, Wv_hf], axis=1).astype(jnp.bfloat16)
    Wp = Wp.astype(jnp.bfloat16)

    qkv_lf, pl_lf = _run_qkv(4, x_bf, scores, labels, w_lf, interpret)
    a_lf = _run_attn(4, qkv_lf, pl_lf, None, interpret)

    qkv_mf, pl_mf = _run_qkv(2, x_bf, scores, labels, w_mf, interpret)
    a_mf = _run_attn(2, qkv_mf, pl_mf, None, interpret)

    qkv_hf, _ = _run_qkv(1, x_bf, scores, labels, w_hf, interpret)
    ng_hf = N // GS
    pl_hf = labels.reshape(B * ng_hf, 1, GS)
    out = _run_attn(1, qkv_hf, pl_hf, (a_mf, a_lf, x, Wp), interpret)
    return out


def kernel(x, labels, scores, Wq_hf, Wk_hf, Wv_hf, Wq_mf, Wk_mf, Wv_mf,
           Wq_lf, Wk_lf, Wv_lf, Wp):
    return _impl(x, labels, scores, Wq_hf, Wk_hf, Wv_hf, Wq_mf, Wk_mf,
                 Wv_mf, Wq_lf, Wk_lf, Wv_lf, Wp)
